# baseline (speedup 1.0000x reference)
import jax
import jax.numpy as jnp
from jax.experimental import pallas as pl

_ROW_BLK = 512


def _nn_kernel(x_ref, c_ref, out_ref):
    x = x_ref[...]
    c = c_ref[...]
    mm = jax.lax.dot_general(
        x, c, (((1,), (1,)), ((), ())),
        preferred_element_type=jnp.float32,
    )
    x_sq = jnp.sum(x * x, axis=1, keepdims=True)
    c_sq = jnp.sum(c * c, axis=1)
    d = (x_sq - 2.0 * mm) + c_sq[None, :]
    idx = jnp.argmin(d, axis=1).astype(jnp.int32)
    out_ref[0, 0, :] = idx


def kernel(x, centroids):
    lead = x.shape[:-1]
    fdim = x.shape[-1]
    flat = x.reshape(-1, fdim)
    n = flat.shape[0]
    blk = _ROW_BLK
    assert n % blk == 0, (n, blk)
    nblk = n // blk
    out = pl.pallas_call(
        _nn_kernel,
        grid=(nblk,),
        in_specs=[
            pl.BlockSpec((blk, fdim), lambda i: (i, 0)),
            pl.BlockSpec(centroids.shape, lambda i: (0, 0)),
        ],
        out_specs=pl.BlockSpec((1, 1, blk), lambda i: (i, 0, 0)),
        out_shape=jax.ShapeDtypeStruct((nblk, 1, blk), jnp.int32),
    )(flat, centroids)
    return out.reshape(lead)
